# baseline (speedup 1.0000x reference)
import functools

import jax
import jax.numpy as jnp
from jax import lax
from jax.experimental import pallas as pl
from jax.experimental.pallas import tpu as pltpu
from jax.experimental.pallas import tpu_sc as plsc

N_NODES = 10000
N_EDGES = 320000
NODE_DIM = 128
MAX_DEG = 256

NB = 10240
HROWS = NB // 128
NSUB = 16
EPW = N_EDGES // NSUB
CHUNKS = EPW // 16
EHALF = EPW // 2


def _sc_degree_body(edge_hbm, hists_hbm, edgebuf, hist, hist2, sem0, sem1):
    cid = lax.axis_index("c")
    sid = lax.axis_index("s")
    erow = 1 - cid
    base = erow * N_EDGES + sid * EPW

    cp0 = pltpu.async_copy(
        edge_hbm.at[pl.ds(base, EHALF)], edgebuf.at[pl.ds(0, EHALF)], sem0)
    cp1 = pltpu.async_copy(
        edge_hbm.at[pl.ds(base + EHALF, EHALF)],
        edgebuf.at[pl.ds(EHALF, EHALF)], sem1)

    zeros16 = jnp.zeros((16,), jnp.int32)
    for i in range(NB // 16):
        hist[pl.ds(i * 16, 16)] = zeros16
        hist2[pl.ds(i * 16, 16)] = zeros16

    ones16 = jnp.ones((16,), jnp.int32)
    UNROLL = 10

    def scatter_body(i, carry):
        for u in range(0, UNROLL, 2):
            idx = edgebuf[pl.ds((i * UNROLL + u) * 16, 16)]
            plsc.addupdate_scatter(hist, [idx], ones16)
            idx2 = edgebuf[pl.ds((i * UNROLL + u + 1) * 16, 16)]
            plsc.addupdate_scatter(hist2, [idx2], ones16)
        return carry

    cp0.wait()
    lax.fori_loop(0, CHUNKS // (2 * UNROLL), scatter_body, 0)
    cp1.wait()
    lax.fori_loop(CHUNKS // (2 * UNROLL), CHUNKS // UNROLL, scatter_body, 0)

    for i in range(NB // 16):
        hist[pl.ds(i * 16, 16)] = (
            hist[pl.ds(i * 16, 16)] + hist2[pl.ds(i * 16, 16)])

    pltpu.sync_copy(hist, hists_hbm.at[pl.ds((cid * NSUB + sid) * NB, NB)])


def _sc_degrees(edge_index):
    mesh = plsc.VectorSubcoreMesh(core_axis_name="c", subcore_axis_name="s")
    f = functools.partial(
        pl.kernel,
        mesh=mesh,
        out_type=jax.ShapeDtypeStruct((2 * NSUB * NB,), jnp.int32),
        scratch_types=[
            pltpu.VMEM((EPW,), jnp.int32),
            pltpu.VMEM((NB,), jnp.int32),
            pltpu.VMEM((NB,), jnp.int32),
            pltpu.SemaphoreType.DMA,
            pltpu.SemaphoreType.DMA,
        ],
        compiler_params=pltpu.CompilerParams(needs_layout_passes=False),
    )(_sc_degree_body)
    return f(edge_index.reshape(-1))


BLK = 1024


def _tree_sum16(a):
    vals = [a[t] for t in range(NSUB)]
    while len(vals) > 1:
        vals = [vals[i] + vals[i + 1] for i in range(0, len(vals), 2)]
    return vals[0]


def _tc_encode_body(h_ref, x_ref, zin_ref, zout_ref, out_ref):
    h = h_ref[...]
    din = jnp.minimum(_tree_sum16(h[0]), MAX_DEG - 1).reshape(BLK)
    dout = jnp.minimum(_tree_sum16(h[1]), MAX_DEG - 1).reshape(BLK)
    iota_t = lax.broadcasted_iota(jnp.int32, (MAX_DEG, BLK), 0)
    oh_in_t = (din[None, :] == iota_t).astype(jnp.float32)
    oh_out_t = (dout[None, :] == iota_t).astype(jnp.float32)
    dn = (((0,), (0,)), ((), ()))
    acc = lax.dot_general(oh_in_t, zin_ref[...], dn,
                          preferred_element_type=jnp.float32)
    acc = acc + lax.dot_general(oh_out_t, zout_ref[...], dn,
                                preferred_element_type=jnp.float32)
    out_ref[...] = x_ref[...] + acc


def _tc_encode(hists, x, z_in, z_out):
    grid = (NB // BLK,)
    return pl.pallas_call(
        _tc_encode_body,
        grid=grid,
        in_specs=[
            pl.BlockSpec((2, NSUB, BLK // 128, 128), lambda i: (0, 0, i, 0)),
            pl.BlockSpec((BLK, NODE_DIM), lambda i: (i, 0)),
            pl.BlockSpec((MAX_DEG, NODE_DIM), lambda i: (0, 0)),
            pl.BlockSpec((MAX_DEG, NODE_DIM), lambda i: (0, 0)),
        ],
        out_specs=pl.BlockSpec((BLK, NODE_DIM), lambda i: (i, 0)),
        out_shape=jax.ShapeDtypeStruct((N_NODES, NODE_DIM), jnp.float32),
    )(hists, x, z_in, z_out)


def kernel(x, edge_index, edge_attr, voronoi_values, centralities, z_in, z_out):
    hists = _sc_degrees(edge_index).reshape(2, NSUB, HROWS, 128)
    return _tc_encode(hists, x, z_in, z_out)


# speedup vs baseline: 1.3315x; 1.3315x over previous
import functools

import jax
import jax.numpy as jnp
from jax import lax
from jax.experimental import pallas as pl
from jax.experimental.pallas import tpu as pltpu
from jax.experimental.pallas import tpu_sc as plsc

N_NODES = 10000
N_EDGES = 320000
NODE_DIM = 128
MAX_DEG = 256

NB = 10240
HROWS = NB // 128
NSUB = 16
EPW = N_EDGES // NSUB
CHUNKS = EPW // 16
EHALF = EPW // 2


def _sc_degree_body(edge_hbm, hists_hbm, edgebuf, hist, hist2, sem0, sem1):
    cid = lax.axis_index("c")
    sid = lax.axis_index("s")
    erow = 1 - cid
    base = erow * N_EDGES + sid * EPW

    cp0 = pltpu.async_copy(
        edge_hbm.at[pl.ds(base, EHALF)], edgebuf.at[pl.ds(0, EHALF)], sem0)
    cp1 = pltpu.async_copy(
        edge_hbm.at[pl.ds(base + EHALF, EHALF)],
        edgebuf.at[pl.ds(EHALF, EHALF)], sem1)

    zeros16 = jnp.zeros((16,), jnp.int32)
    for i in range(NB // 16):
        hist[pl.ds(i * 16, 16)] = zeros16

    ones16 = jnp.ones((16,), jnp.int32)

    cp0.wait()

    @plsc.parallel_loop(0, CHUNKS // 2, 1, unroll=10)
    def _(i):
        idx = edgebuf[pl.ds(i * 16, 16)]
        plsc.addupdate_scatter(hist, [idx], ones16)

    cp1.wait()

    @plsc.parallel_loop(CHUNKS // 2, CHUNKS, 1, unroll=10)
    def _(i):
        idx = edgebuf[pl.ds(i * 16, 16)]
        plsc.addupdate_scatter(hist, [idx], ones16)

    pltpu.sync_copy(hist, hists_hbm.at[pl.ds((cid * NSUB + sid) * NB, NB)])


def _sc_degrees(edge_index):
    mesh = plsc.VectorSubcoreMesh(core_axis_name="c", subcore_axis_name="s")
    f = functools.partial(
        pl.kernel,
        mesh=mesh,
        out_type=jax.ShapeDtypeStruct((2 * NSUB * NB,), jnp.int32),
        scratch_types=[
            pltpu.VMEM((EPW,), jnp.int32),
            pltpu.VMEM((NB,), jnp.int32),
            pltpu.VMEM((NB,), jnp.int32),
            pltpu.SemaphoreType.DMA,
            pltpu.SemaphoreType.DMA,
        ],
        compiler_params=pltpu.CompilerParams(needs_layout_passes=False),
    )(_sc_degree_body)
    return f(edge_index.reshape(-1))


BLK = 1024


def _tree_sum16(a):
    vals = [a[t] for t in range(NSUB)]
    while len(vals) > 1:
        vals = [vals[i] + vals[i + 1] for i in range(0, len(vals), 2)]
    return vals[0]


def _tc_encode_body(h_ref, x_ref, zin_ref, zout_ref, out_ref):
    h = h_ref[...]
    din = jnp.minimum(_tree_sum16(h[0]), MAX_DEG - 1).reshape(BLK)
    dout = jnp.minimum(_tree_sum16(h[1]), MAX_DEG - 1).reshape(BLK)
    iota_t = lax.broadcasted_iota(jnp.int32, (MAX_DEG, BLK), 0)
    oh_in_t = (din[None, :] == iota_t).astype(jnp.float32)
    oh_out_t = (dout[None, :] == iota_t).astype(jnp.float32)
    dn = (((0,), (0,)), ((), ()))
    acc = lax.dot_general(oh_in_t, zin_ref[...], dn,
                          preferred_element_type=jnp.float32)
    acc = acc + lax.dot_general(oh_out_t, zout_ref[...], dn,
                                preferred_element_type=jnp.float32)
    out_ref[...] = x_ref[...] + acc


def _tc_encode(hists, x, z_in, z_out):
    grid = (NB // BLK,)
    return pl.pallas_call(
        _tc_encode_body,
        grid=grid,
        in_specs=[
            pl.BlockSpec((2, NSUB, BLK // 128, 128), lambda i: (0, 0, i, 0)),
            pl.BlockSpec((BLK, NODE_DIM), lambda i: (i, 0)),
            pl.BlockSpec((MAX_DEG, NODE_DIM), lambda i: (0, 0)),
            pl.BlockSpec((MAX_DEG, NODE_DIM), lambda i: (0, 0)),
        ],
        out_specs=pl.BlockSpec((BLK, NODE_DIM), lambda i: (i, 0)),
        out_shape=jax.ShapeDtypeStruct((N_NODES, NODE_DIM), jnp.float32),
    )(hists, x, z_in, z_out)


def kernel(x, edge_index, edge_attr, voronoi_values, centralities, z_in, z_out):
    hists = _sc_degrees(edge_index).reshape(2, NSUB, HROWS, 128)
    return _tc_encode(hists, x, z_in, z_out)


# speedup vs baseline: 1.4505x; 1.0893x over previous
import functools

import jax
import jax.numpy as jnp
from jax import lax
from jax.experimental import pallas as pl
from jax.experimental.pallas import tpu as pltpu
from jax.experimental.pallas import tpu_sc as plsc

N_NODES = 10000
N_EDGES = 320000
NODE_DIM = 128
MAX_DEG = 256

NB = 10240
HROWS = NB // 128
NSUB = 16
EPW = N_EDGES // NSUB
CHUNKS = EPW // 16
EHALF = EPW // 2


def _sc_degree_body(edge_hbm, hists_hbm, edgebuf, hist, hist2, sem0, sem1):
    cid = lax.axis_index("c")
    sid = lax.axis_index("s")
    erow = 1 - cid
    base = erow * N_EDGES + sid * EPW

    cp0 = pltpu.async_copy(
        edge_hbm.at[pl.ds(base, EHALF)], edgebuf.at[pl.ds(0, EHALF)], sem0)
    cp1 = pltpu.async_copy(
        edge_hbm.at[pl.ds(base + EHALF, EHALF)],
        edgebuf.at[pl.ds(EHALF, EHALF)], sem1)

    zeros16 = jnp.zeros((16,), jnp.int32)

    @plsc.parallel_loop(0, NB // 16, 1, unroll=8)
    def _(i):
        hist[pl.ds(i * 16, 16)] = zeros16

    ones16 = jnp.ones((16,), jnp.int32)

    cp0.wait()

    @plsc.parallel_loop(0, CHUNKS // 2, 1, unroll=10)
    def _(i):
        idx = edgebuf[pl.ds(i * 16, 16)]
        plsc.addupdate_scatter(hist, [idx], ones16)

    cp1.wait()

    @plsc.parallel_loop(CHUNKS // 2, CHUNKS, 1, unroll=10)
    def _(i):
        idx = edgebuf[pl.ds(i * 16, 16)]
        plsc.addupdate_scatter(hist, [idx], ones16)

    pltpu.sync_copy(hist, hists_hbm.at[pl.ds((cid * NSUB + sid) * NB, NB)])


def _sc_degrees(edge_index):
    mesh = plsc.VectorSubcoreMesh(core_axis_name="c", subcore_axis_name="s")
    f = functools.partial(
        pl.kernel,
        mesh=mesh,
        out_type=jax.ShapeDtypeStruct((2 * NSUB * NB,), jnp.int32),
        scratch_types=[
            pltpu.VMEM((EPW,), jnp.int32),
            pltpu.VMEM((NB,), jnp.int32),
            pltpu.VMEM((NB,), jnp.int32),
            pltpu.SemaphoreType.DMA,
            pltpu.SemaphoreType.DMA,
        ],
        compiler_params=pltpu.CompilerParams(needs_layout_passes=False),
    )(_sc_degree_body)
    return f(edge_index.reshape(-1))


BLK = 2048


def _tree_sum16(a):
    vals = [a[t] for t in range(NSUB)]
    while len(vals) > 1:
        vals = [vals[i] + vals[i + 1] for i in range(0, len(vals), 2)]
    return vals[0]


def _tc_encode_body(h_ref, x_ref, zin_ref, zout_ref, out_ref):
    h = h_ref[...]
    din = jnp.minimum(_tree_sum16(h[0]), MAX_DEG - 1).reshape(BLK)
    dout = jnp.minimum(_tree_sum16(h[1]), MAX_DEG - 1).reshape(BLK)
    iota_t = lax.broadcasted_iota(jnp.int32, (MAX_DEG, BLK), 0)
    oh_in_t = (din[None, :] == iota_t).astype(jnp.float32)
    oh_out_t = (dout[None, :] == iota_t).astype(jnp.float32)
    dn = (((0,), (0,)), ((), ()))
    acc = lax.dot_general(oh_in_t, zin_ref[...], dn,
                          preferred_element_type=jnp.float32)
    acc = acc + lax.dot_general(oh_out_t, zout_ref[...], dn,
                                preferred_element_type=jnp.float32)
    out_ref[...] = x_ref[...] + acc


def _tc_encode(hists, x, z_in, z_out):
    grid = (NB // BLK,)
    return pl.pallas_call(
        _tc_encode_body,
        grid=grid,
        in_specs=[
            pl.BlockSpec((2, NSUB, BLK // 128, 128), lambda i: (0, 0, i, 0)),
            pl.BlockSpec((BLK, NODE_DIM), lambda i: (i, 0)),
            pl.BlockSpec((MAX_DEG, NODE_DIM), lambda i: (0, 0)),
            pl.BlockSpec((MAX_DEG, NODE_DIM), lambda i: (0, 0)),
        ],
        out_specs=pl.BlockSpec((BLK, NODE_DIM), lambda i: (i, 0)),
        out_shape=jax.ShapeDtypeStruct((N_NODES, NODE_DIM), jnp.float32),
    )(hists, x, z_in, z_out)


def kernel(x, edge_index, edge_attr, voronoi_values, centralities, z_in, z_out):
    hists = _sc_degrees(edge_index).reshape(2, NSUB, HROWS, 128)
    return _tc_encode(hists, x, z_in, z_out)
